# baseline (speedup 1.0000x reference)
import functools

import jax
import jax.numpy as jnp
from jax import lax
from jax.experimental import pallas as pl
from jax.experimental.pallas import tpu as pltpu
from jax.experimental.pallas import tpu_sc as plsc

VOCAB = 1000000
EMBED = 32
BATCH = 16384

_info = plsc.get_sparse_core_info()
_NC, _NS, _L = _info.num_cores, _info.num_subcores, _info.num_lanes
_NW = _NC * _NS
_BPW = BATCH // _NW


def _sc_kernel(idx_t_hbm, idx_c_hbm, table_hbm, w_hbm, b_hbm, out_hbm,
               idx_t_v, idx_c_v, t_v, c_v, out_v, w_v, b_v, sem_t, sem_c):
    wid = lax.axis_index("s") * _NC + lax.axis_index("c")
    base = wid * _BPW
    pltpu.sync_copy(idx_t_hbm.at[pl.ds(base, _BPW)], idx_t_v)
    pltpu.sync_copy(idx_c_hbm.at[pl.ds(base, _BPW)], idx_c_v)
    pltpu.sync_copy(w_hbm, w_v)
    pltpu.sync_copy(b_hbm, b_v)
    cp_t = pltpu.async_copy(table_hbm.at[idx_t_v], t_v, sem_t)
    cp_c = pltpu.async_copy(table_hbm.at[idx_c_v], c_v, sem_c)
    cp_t.wait()
    cp_c.wait()

    wv = w_v[...]
    bv = b_v[...]
    lanes = lax.iota(jnp.int32, 16)

    def body(g, carry):
        rows = jnp.int32(g) * 16 + lanes
        acc = jnp.zeros((16,), jnp.float32)
        for e in range(EMBED):
            col = jnp.full((16,), e, jnp.int32)
            tv = plsc.load_gather(t_v, [rows, col])
            cv = plsc.load_gather(c_v, [rows, col])
            acc = acc + tv * cv
        z = acc * wv + bv
        out_v[pl.ds(g * 16, 16)] = 1.0 / (1.0 + jnp.exp(-z))
        return carry

    lax.fori_loop(0, _BPW // 16, body, 0)
    pltpu.sync_copy(out_v, out_hbm.at[pl.ds(base, _BPW)])


@functools.partial(
    pl.kernel,
    out_type=jax.ShapeDtypeStruct((BATCH,), jnp.float32),
    mesh=plsc.VectorSubcoreMesh(core_axis_name="c", subcore_axis_name="s"),
    compiler_params=pltpu.CompilerParams(needs_layout_passes=False, use_tc_tiling_on_sc=False),
    scratch_types=[
        pltpu.VMEM((_BPW,), jnp.int32),
        pltpu.VMEM((_BPW,), jnp.int32),
        pltpu.VMEM((_BPW, EMBED), jnp.float32),
        pltpu.VMEM((_BPW, EMBED), jnp.float32),
        pltpu.VMEM((_BPW,), jnp.float32),
        pltpu.VMEM((16,), jnp.float32),
        pltpu.VMEM((16,), jnp.float32),
        pltpu.SemaphoreType.DMA,
        pltpu.SemaphoreType.DMA,
    ],
)
def _context_model_sc(idx_t_hbm, idx_c_hbm, table_hbm, w_hbm, b_hbm, out_hbm,
                      idx_t_v, idx_c_v, t_v, c_v, out_v, w_v, b_v,
                      sem_t, sem_c):
    _sc_kernel(idx_t_hbm, idx_c_hbm, table_hbm, w_hbm, b_hbm, out_hbm,
               idx_t_v, idx_c_v, t_v, c_v, out_v, w_v, b_v, sem_t, sem_c)


def kernel(inputs, table, W, b):
    idx_t = inputs[:, 0].astype(jnp.int32)
    idx_c = inputs[:, 1].astype(jnp.int32)
    w16 = jnp.full((16,), W[0, 0], dtype=jnp.float32)
    b16 = jnp.full((16,), b[0], dtype=jnp.float32)
    out = _context_model_sc(idx_t, idx_c, table, w16, b16)
    return out.reshape(BATCH, 1)
